# baseline (speedup 1.0000x reference)
import jax
import jax.numpy as jnp
from jax import lax
from jax.experimental import pallas as pl
from jax.experimental.pallas import tpu as pltpu
from jax.experimental.pallas import tpu_sc as plsc

_VOCAB = 1000000
_D = 128
_CTX = 200
_HID = 128

_PER_W = 8
_ACTIVE = _CTX // _PER_W

_B0 = 880000
_C = 16000
_NCH = _B0 // _C
_K = 4
_KO = 4

_SCN = _VOCAB - _B0
_SCW = 30
_WROWS = _SCN // _SCW
_BS = 400
_NB = _WROWS // _BS
_OSC_CW = 16000
_OSC_R = 8

_NOUT = (_VOCAB + _C - 1) // _C


def _sc_gather_body(idx_hbm, emb_hbm, out_hbm, idx_v, rows_v, part_v, sem):
    wid = lax.axis_index("s") * 2 + lax.axis_index("c")

    @pl.when(wid < _ACTIVE)
    def _():
        base = wid * _PER_W
        pltpu.sync_copy(idx_hbm.at[pl.ds(base, _PER_W)], idx_v)
        pltpu.async_copy(emb_hbm.at[idx_v], rows_v, sem).wait()
        for c in range(_D // 16):
            acc = jnp.zeros((16,), jnp.float32)
            for r in range(_PER_W):
                acc = acc + rows_v[r, pl.ds(c * 16, 16)]
            part_v[pl.ds(c * 16, 16)] = acc
        pltpu.sync_copy(part_v, out_hbm.at[wid])


def _sc_gather_sum(x, emb):
    f = pl.kernel(
        _sc_gather_body,
        out_type=jax.ShapeDtypeStruct((_ACTIVE, _D), jnp.float32),
        mesh=plsc.VectorSubcoreMesh(core_axis_name="c", subcore_axis_name="s"),
        scratch_types=[
            pltpu.VMEM((_PER_W,), jnp.int32),
            pltpu.VMEM((_PER_W, _D), jnp.float32),
            pltpu.VMEM((_D,), jnp.float32),
            pltpu.SemaphoreType.DMA,
        ],
    )
    return f(x, emb)


def _h_body(parts, wp, w1, b1, h_ref):
    s = jnp.sum(parts[...], axis=0, keepdims=True)
    p = lax.dot_general(s, wp[...], (((1,), (1,)), ((), ())),
                        preferred_element_type=jnp.float32)
    hh = lax.dot_general(p, w1[...], (((1,), (1,)), ((), ())),
                         preferred_element_type=jnp.float32) + b1[...]
    h_ref[...] = jnp.maximum(hh, 0.0)


def _tc_h(parts, wp, w1, b1):
    return pl.pallas_call(
        _h_body,
        out_shape=jax.ShapeDtypeStruct((1, _D), jnp.float32),
    )(parts, wp, w1, b1)


def _logits_body(h_in, w2_hbm, b2_hbm, o_hbm, st_ref,
                 w2buf, b2buf, obuf, in_sems, b2_sem, out_sems, m_scr, s_scr):
    h = h_in[...].astype(jnp.bfloat16)
    m_scr[0] = -jnp.inf
    s_scr[0] = 0.0

    def _start_in(c, slot):
        pltpu.make_async_copy(
            w2_hbm.at[pl.ds(c * _C, _C), :], w2buf.at[slot],
            in_sems.at[slot]).start()

    b2_cp = pltpu.make_async_copy(b2_hbm, b2buf, b2_sem)
    b2_cp.start()
    for c in range(_K - 1):
        _start_in(c, c)
    b2_cp.wait()

    def _step(i, _):
        slot = lax.rem(i, _K)
        nxt = i + _K - 1

        @pl.when(nxt < _NCH)
        def _():
            _start_in(nxt, lax.rem(nxt, _K))

        pltpu.make_async_copy(
            w2_hbm.at[pl.ds(i * _C, _C), :], w2buf.at[slot],
            in_sems.at[slot]).wait()

        o_t = lax.dot_general(h, w2buf[slot].astype(jnp.bfloat16),
                              (((1,), (1,)), ((), ())),
                              preferred_element_type=jnp.float32) + b2buf[pl.ds(i, 1), :]
        m_old = m_scr[0]
        m_new = jnp.maximum(m_old, jnp.max(o_t))
        s_scr[0] = s_scr[0] * jnp.exp(m_old - m_new) + jnp.sum(jnp.exp(o_t - m_new))
        m_scr[0] = m_new

        oslot = lax.rem(i, _KO)

        @pl.when(i >= _KO)
        def _():
            pltpu.make_async_copy(
                obuf.at[oslot], o_hbm.at[pl.ds(i - _KO, 1), :],
                out_sems.at[oslot]).wait()

        obuf[oslot, 0] = o_t
        pltpu.make_async_copy(
            obuf.at[oslot], o_hbm.at[pl.ds(i, 1)],
            out_sems.at[oslot]).start()
        return 0

    lax.fori_loop(0, _NCH, _step, 0)

    for d in range(_KO):
        c = _NCH - _KO + d
        pltpu.make_async_copy(
            obuf.at[c % _KO], o_hbm.at[pl.ds(c, 1)],
            out_sems.at[c % _KO]).wait()

    st_ref[0, 0] = m_scr[0]
    st_ref[0, 1] = s_scr[0]


def _tc_logits(h, w2, b2_tc):
    return pl.pallas_call(
        _logits_body,
        in_specs=[
            pl.BlockSpec(memory_space=pltpu.VMEM),
            pl.BlockSpec(memory_space=pl.ANY),
            pl.BlockSpec(memory_space=pl.ANY),
        ],
        out_specs=[
            pl.BlockSpec(memory_space=pl.ANY),
            pl.BlockSpec(memory_space=pltpu.SMEM),
        ],
        out_shape=[
            jax.ShapeDtypeStruct((_NCH, 1, _C), jnp.float32),
            jax.ShapeDtypeStruct((1, 2), jnp.float32),
        ],
        scratch_shapes=[
            pltpu.VMEM((_K, _C, _D), jnp.float32),
            pltpu.VMEM((_NCH, _C), jnp.float32),
            pltpu.VMEM((_KO, 1, 1, _C), jnp.float32),
            pltpu.SemaphoreType.DMA((_K,)),
            pltpu.SemaphoreType.DMA,
            pltpu.SemaphoreType.DMA((_KO,)),
            pltpu.SMEM((1,), jnp.float32),
            pltpu.SMEM((1,), jnp.float32),
        ],
    )(h, w2, b2_tc)


def _sc_mv_body(h_hbm, w2_hbm, b2_hbm, o_hbm, st_hbm,
                h_v, blk0, blk1, b2v0, b2v1, o_blk, st_v,
                sem0, sem1, semb0, semb1):
    w = lax.axis_index("s") * 2 + lax.axis_index("c")

    @pl.when(w < _SCW)
    def _():
        base = _B0 + w * _WROWS
        pltpu.sync_copy(h_hbm, h_v)
        obase = w * _WROWS

        blks = (blk0, blk1)
        sems = (sem0, sem1)
        b2vs = (b2v0, b2v1)
        sembs = (semb0, semb1)

        def _start(b):
            s = b % 2
            pltpu.async_copy(
                w2_hbm.at[pl.ds(base + b * _BS, _BS), :], blks[s], sems[s])
            pltpu.async_copy(
                b2_hbm.at[pl.ds(base + b * _BS, _BS)], b2vs[s], sembs[s])

        _start(0)
        mvec = jnp.full((16,), -jnp.inf, jnp.float32)
        svec = jnp.zeros((16,), jnp.float32)

        for b in range(_NB):
            s = b % 2
            if b + 1 < _NB:
                _start(b + 1)
            pltpu.make_async_copy(
                w2_hbm.at[pl.ds(base + b * _BS, _BS), :], blks[s],
                sems[s]).wait()
            pltpu.make_async_copy(
                b2_hbm.at[pl.ds(base + b * _BS, _BS)], b2vs[s],
                sembs[s]).wait()
            cur = blks[s]
            b2c = b2vs[s]
            hcs = [h_v[0, pl.ds(c * 16, 16)] for c in range(8)]
            for gg in range(_BS // 16):
                o_blk[pl.ds(gg * 16, 16)] = jnp.zeros((16,), jnp.float32)

            def _group(g, carry):
                mv, sv = carry
                base_r = g * 16
                for r16 in range(16):
                    r = base_r + r16
                    acc = cur[r, pl.ds(0, 16)] * hcs[0]
                    for c in range(1, 8):
                        acc = acc + cur[r, pl.ds(c * 16, 16)] * hcs[c]
                    plsc.addupdate_scatter(
                        o_blk, [jnp.full((16,), r, jnp.int32)], acc)
                gv = o_blk[pl.ds(base_r, 16)] + b2c[pl.ds(base_r, 16)]
                o_blk[pl.ds(base_r, 16)] = gv
                mn = jnp.maximum(mv, gv)
                sv = sv * jnp.exp(mv - mn) + jnp.exp(gv - mn)
                return (mn, sv)

            mvec, svec = lax.fori_loop(0, _BS // 16, _group, (mvec, svec))
            pltpu.sync_copy(o_blk, o_hbm.at[pl.ds(obase + b * _BS, _BS)])

        st_v[0, pl.ds(0, 16)] = mvec
        st_v[0, pl.ds(16, 16)] = svec
        pltpu.sync_copy(st_v, st_hbm.at[pl.ds(w, 1), :])


def _sc_matvec(h, w2, b2):
    f = pl.kernel(
        _sc_mv_body,
        out_type=[
            jax.ShapeDtypeStruct((_OSC_R * _OSC_CW,), jnp.float32),
            jax.ShapeDtypeStruct((_SCW, 128), jnp.float32),
        ],
        mesh=plsc.VectorSubcoreMesh(core_axis_name="c", subcore_axis_name="s"),
        compiler_params=pltpu.CompilerParams(needs_layout_passes=False),
        scratch_types=[
            pltpu.VMEM((1, _D), jnp.float32),
            pltpu.VMEM((_BS, _D), jnp.float32),
            pltpu.VMEM((_BS, _D), jnp.float32),
            pltpu.VMEM((_BS,), jnp.float32),
            pltpu.VMEM((_BS,), jnp.float32),
            pltpu.VMEM((_BS,), jnp.float32),
            pltpu.VMEM((1, 128), jnp.float32),
            pltpu.SemaphoreType.DMA,
            pltpu.SemaphoreType.DMA,
            pltpu.SemaphoreType.DMA,
            pltpu.SemaphoreType.DMA,
        ],
    )
    return f(h, w2, b2)


def _norm_body(otc, osc, st_tc, st_sc, out_ref, lse_scr):
    j = pl.program_id(0)

    @pl.when(j == 0)
    def _():
        m1 = st_tc[0, 0]
        s1 = st_tc[0, 1]
        mvs = st_sc[:, 0:16]
        svs = st_sc[:, 16:32]

        mg = jnp.maximum(m1, jnp.max(mvs))
        sg = s1 * jnp.exp(m1 - mg) + jnp.sum(svs * jnp.exp(mvs - mg))
        lse_scr[0] = mg + jnp.log(sg)

    lse = lse_scr[0]

    @pl.when(j < _NCH)
    def _():
        out_ref[...] = otc[0] - lse

    @pl.when(j >= _NCH)
    def _():
        out_ref[...] = osc[0] - lse


def _tc_norm(otc, osc, st_tc, st_sc):
    return pl.pallas_call(
        _norm_body,
        grid=(_NOUT,),
        in_specs=[
            pl.BlockSpec((1, 1, _C), lambda j: (jnp.minimum(j, _NCH - 1), 0, 0)),
            pl.BlockSpec((1, 1, _OSC_CW),
                         lambda j: (jnp.maximum(j - _NCH, 0), 0, 0)),
            pl.BlockSpec(memory_space=pltpu.SMEM),
            pl.BlockSpec((_SCW, 128), lambda j: (0, 0)),
        ],
        out_specs=pl.BlockSpec((1, _C), lambda j: (0, j)),
        out_shape=jax.ShapeDtypeStruct((1, _VOCAB), jnp.float32),
        scratch_shapes=[pltpu.SMEM((1,), jnp.float32)],
    )(otc, osc, st_tc, st_sc)


def kernel(x, emb, W_proj, W1, b1, W2, b2):
    x = x.astype(jnp.int32)
    parts = _sc_gather_sum(x, emb)
    h = _tc_h(parts, W_proj, W1, b1.reshape(1, _HID))
    b2_tc = b2[:_B0].reshape(_NCH, _C)
    osc_flat, st_sc = _sc_matvec(h, W2, b2)
    otc, st_tc = _tc_logits(h, W2, b2_tc)
    return _tc_norm(otc, osc_flat.reshape(_OSC_R, 1, _OSC_CW), st_tc, st_sc)


# speedup vs baseline: 1.0580x; 1.0580x over previous
import jax
import jax.numpy as jnp
from jax import lax
from jax.experimental import pallas as pl
from jax.experimental.pallas import tpu as pltpu
from jax.experimental.pallas import tpu_sc as plsc

_VOCAB = 1000000
_D = 128
_CTX = 200
_HID = 128

_PER_W = 8
_ACTIVE = _CTX // _PER_W

_T = 32768
_NSTEPS = (_VOCAB + _T - 1) // _T

_C = 8000
_NCH = _VOCAB // _C
_K = 4
_KO = 4


def _sc_body(idx_hbm, emb_hbm, out_hbm, idx_v, rows_v, part_v, sem):
    wid = lax.axis_index("s") * 2 + lax.axis_index("c")

    @pl.when(wid < _ACTIVE)
    def _():
        base = wid * _PER_W
        pltpu.sync_copy(idx_hbm.at[pl.ds(base, _PER_W)], idx_v)
        pltpu.async_copy(emb_hbm.at[idx_v], rows_v, sem).wait()
        for c in range(_D // 16):
            acc = jnp.zeros((16,), jnp.float32)
            for r in range(_PER_W):
                acc = acc + rows_v[r, pl.ds(c * 16, 16)]
            part_v[pl.ds(c * 16, 16)] = acc
        pltpu.sync_copy(part_v, out_hbm.at[wid])


def _sc_gather_sum(x, emb):
    f = pl.kernel(
        _sc_body,
        out_type=jax.ShapeDtypeStruct((_ACTIVE, _D), jnp.float32),
        mesh=plsc.VectorSubcoreMesh(core_axis_name="c", subcore_axis_name="s"),
        scratch_types=[
            pltpu.VMEM((_PER_W,), jnp.int32),
            pltpu.VMEM((_PER_W, _D), jnp.float32),
            pltpu.VMEM((_D,), jnp.float32),
            pltpu.SemaphoreType.DMA,
        ],
    )
    return f(x, emb)


def _logits_body(parts, wp, w1, b1, w2_hbm, b2_hbm, o_hbm, lse_ref,
                 w2buf, b2buf, obuf, in_sems, ob_sems, out_sems,
                 m_scr, s_scr):
    s = jnp.sum(parts[...], axis=0, keepdims=True)
    p = lax.dot_general(s, wp[...], (((1,), (1,)), ((), ())),
                        preferred_element_type=jnp.float32)
    h = lax.dot_general(p, w1[...], (((1,), (1,)), ((), ())),
                        preferred_element_type=jnp.float32) + b1[...]
    h = jnp.maximum(h, 0.0).astype(jnp.bfloat16)
    m_scr[0] = -jnp.inf
    s_scr[0] = 0.0

    def _start_in(c, slot):
        pltpu.make_async_copy(
            w2_hbm.at[pl.ds(c * _C, _C), :], w2buf.at[slot],
            in_sems.at[slot]).start()
        pltpu.make_async_copy(
            b2_hbm.at[pl.ds(c, 1), :], b2buf.at[slot],
            ob_sems.at[slot]).start()

    for c in range(_K - 1):
        _start_in(c, c)

    def _step(i, _):
        slot = lax.rem(i, _K)
        nxt = i + _K - 1

        @pl.when(nxt < _NCH)
        def _():
            _start_in(nxt, lax.rem(nxt, _K))

        pltpu.make_async_copy(
            w2_hbm.at[pl.ds(i * _C, _C), :], w2buf.at[slot],
            in_sems.at[slot]).wait()
        pltpu.make_async_copy(
            b2_hbm.at[pl.ds(i, 1), :], b2buf.at[slot],
            ob_sems.at[slot]).wait()

        o_t = lax.dot_general(h, w2buf[slot].astype(jnp.bfloat16),
                              (((1,), (1,)), ((), ())),
                              preferred_element_type=jnp.float32) + b2buf[slot]
        m_old = m_scr[0]
        m_new = jnp.maximum(m_old, jnp.max(o_t))
        s_scr[0] = s_scr[0] * jnp.exp(m_old - m_new) + jnp.sum(jnp.exp(o_t - m_new))
        m_scr[0] = m_new

        oslot = lax.rem(i, _KO)

        @pl.when(i >= _KO)
        def _():
            pltpu.make_async_copy(
                obuf.at[oslot], o_hbm.at[pl.ds(i - _KO, 1), :],
                out_sems.at[oslot]).wait()

        obuf[oslot] = o_t
        pltpu.make_async_copy(
            obuf.at[oslot], o_hbm.at[pl.ds(i, 1), :],
            out_sems.at[oslot]).start()
        return 0

    lax.fori_loop(0, _NCH, _step, 0)

    for d in range(_KO):
        c = _NCH - _KO + d
        pltpu.make_async_copy(
            obuf.at[c % _KO], o_hbm.at[pl.ds(c, 1), :],
            out_sems.at[c % _KO]).wait()

    lse_ref[0, 0] = m_scr[0] + jnp.log(s_scr[0])


def _tc_logits(parts, wp, w1, b1, w2, b2):
    return pl.pallas_call(
        _logits_body,
        in_specs=[
            pl.BlockSpec(memory_space=pltpu.VMEM),
            pl.BlockSpec(memory_space=pltpu.VMEM),
            pl.BlockSpec(memory_space=pltpu.VMEM),
            pl.BlockSpec(memory_space=pltpu.VMEM),
            pl.BlockSpec(memory_space=pl.ANY),
            pl.BlockSpec(memory_space=pl.ANY),
        ],
        out_specs=[
            pl.BlockSpec(memory_space=pl.ANY),
            pl.BlockSpec(memory_space=pltpu.SMEM),
        ],
        out_shape=[
            jax.ShapeDtypeStruct((_NCH, _C), jnp.float32),
            jax.ShapeDtypeStruct((1, 1), jnp.float32),
        ],
        scratch_shapes=[
            pltpu.VMEM((_K, _C, _D), jnp.float32),
            pltpu.VMEM((_K, 1, _C), jnp.float32),
            pltpu.VMEM((_KO, 1, _C), jnp.float32),
            pltpu.SemaphoreType.DMA((_K,)),
            pltpu.SemaphoreType.DMA((_K,)),
            pltpu.SemaphoreType.DMA((_KO,)),
            pltpu.SMEM((1,), jnp.float32),
            pltpu.SMEM((1,), jnp.float32),
        ],
    )(parts, wp, w1, b1, w2, b2)


def _norm_step(o_ref, lse_ref, out_ref):
    out_ref[...] = o_ref[...] - lse_ref[0, 0]


def _tc_norm(o, lse):
    return pl.pallas_call(
        _norm_step,
        grid=(_NSTEPS,),
        in_specs=[
            pl.BlockSpec((1, _T), lambda i: (0, i)),
            pl.BlockSpec(memory_space=pltpu.SMEM),
        ],
        out_specs=pl.BlockSpec((1, _T), lambda i: (0, i)),
        out_shape=jax.ShapeDtypeStruct((1, _VOCAB), jnp.float32),
    )(o, lse)


def kernel(x, emb, W_proj, W1, b1, W2, b2):
    x = x.astype(jnp.int32)
    parts = _sc_gather_sum(x, emb)
    o2, lse = _tc_logits(parts, W_proj, W1,
                         b1.reshape(1, _HID), W2, b2.reshape(_NCH, _C))
    return _tc_norm(o2.reshape(1, _VOCAB), lse)


# speedup vs baseline: 1.0594x; 1.0014x over previous
import jax
import jax.numpy as jnp
from jax import lax
from jax.experimental import pallas as pl
from jax.experimental.pallas import tpu as pltpu
from jax.experimental.pallas import tpu_sc as plsc

_VOCAB = 1000000
_D = 128
_CTX = 200
_HID = 128

_PER_W = 8
_ACTIVE = _CTX // _PER_W

_T = 32768
_NSTEPS = (_VOCAB + _T - 1) // _T

_C = 8000
_NCH = _VOCAB // _C
_K = 4
_KO = 4


def _sc_body(idx_hbm, emb_hbm, out_hbm, idx_v, rows_v, part_v, sem):
    wid = lax.axis_index("s") * 2 + lax.axis_index("c")

    @pl.when(wid < _ACTIVE)
    def _():
        base = wid * _PER_W
        pltpu.sync_copy(idx_hbm.at[pl.ds(base, _PER_W)], idx_v)
        pltpu.async_copy(emb_hbm.at[idx_v], rows_v, sem).wait()
        for c in range(_D // 16):
            acc = jnp.zeros((16,), jnp.float32)
            for r in range(_PER_W):
                acc = acc + rows_v[r, pl.ds(c * 16, 16)]
            part_v[pl.ds(c * 16, 16)] = acc
        pltpu.sync_copy(part_v, out_hbm.at[wid])


def _sc_gather_sum(x, emb):
    f = pl.kernel(
        _sc_body,
        out_type=jax.ShapeDtypeStruct((_ACTIVE, _D), jnp.float32),
        mesh=plsc.VectorSubcoreMesh(core_axis_name="c", subcore_axis_name="s"),
        scratch_types=[
            pltpu.VMEM((_PER_W,), jnp.int32),
            pltpu.VMEM((_PER_W, _D), jnp.float32),
            pltpu.VMEM((_D,), jnp.float32),
            pltpu.SemaphoreType.DMA,
        ],
    )
    return f(x, emb)


def _logits_body(parts, wp, w1, b1, w2_hbm, b2_hbm, o_hbm, lse_ref,
                 w2buf, b2buf, obuf, in_sems, ob_sems, out_sems,
                 m_scr, s_scr):
    s = jnp.sum(parts[...], axis=0, keepdims=True)
    p = lax.dot_general(s, wp[...], (((1,), (1,)), ((), ())),
                        preferred_element_type=jnp.float32)
    h = lax.dot_general(p, w1[...], (((1,), (1,)), ((), ())),
                        preferred_element_type=jnp.float32) + b1[...]
    h = jnp.maximum(h, 0.0).astype(jnp.bfloat16)
    m_scr[0] = -jnp.inf
    s_scr[0] = 0.0

    def _start_in(c, slot):
        pltpu.make_async_copy(
            w2_hbm.at[pl.ds(c * _C, _C), :], w2buf.at[slot],
            in_sems.at[slot]).start()

    b2_cp = pltpu.make_async_copy(b2_hbm, b2buf, ob_sems)
    b2_cp.start()
    for c in range(_K - 1):
        _start_in(c, c)
    b2_cp.wait()

    def _step(i, _):
        slot = lax.rem(i, _K)
        nxt = i + _K - 1

        @pl.when(nxt < _NCH)
        def _():
            _start_in(nxt, lax.rem(nxt, _K))

        pltpu.make_async_copy(
            w2_hbm.at[pl.ds(i * _C, _C), :], w2buf.at[slot],
            in_sems.at[slot]).wait()

        o_t = lax.dot_general(h, w2buf[slot].astype(jnp.bfloat16),
                              (((1,), (1,)), ((), ())),
                              preferred_element_type=jnp.float32) + b2buf[pl.ds(i, 1), :]
        m_old = m_scr[0]
        m_new = jnp.maximum(m_old, jnp.max(o_t))
        s_scr[0] = s_scr[0] * jnp.exp(m_old - m_new) + jnp.sum(jnp.exp(o_t - m_new))
        m_scr[0] = m_new

        oslot = lax.rem(i, _KO)

        @pl.when(i >= _KO)
        def _():
            pltpu.make_async_copy(
                obuf.at[oslot], o_hbm.at[pl.ds(i - _KO, 1), :],
                out_sems.at[oslot]).wait()

        obuf[oslot] = o_t
        pltpu.make_async_copy(
            obuf.at[oslot], o_hbm.at[pl.ds(i, 1), :],
            out_sems.at[oslot]).start()
        return 0

    lax.fori_loop(0, _NCH, _step, 0)

    for d in range(_KO):
        c = _NCH - _KO + d
        pltpu.make_async_copy(
            obuf.at[c % _KO], o_hbm.at[pl.ds(c, 1), :],
            out_sems.at[c % _KO]).wait()

    lse_ref[0, 0] = m_scr[0] + jnp.log(s_scr[0])


def _tc_logits(parts, wp, w1, b1, w2, b2):
    return pl.pallas_call(
        _logits_body,
        in_specs=[
            pl.BlockSpec(memory_space=pltpu.VMEM),
            pl.BlockSpec(memory_space=pltpu.VMEM),
            pl.BlockSpec(memory_space=pltpu.VMEM),
            pl.BlockSpec(memory_space=pltpu.VMEM),
            pl.BlockSpec(memory_space=pl.ANY),
            pl.BlockSpec(memory_space=pl.ANY),
        ],
        out_specs=[
            pl.BlockSpec(memory_space=pl.ANY),
            pl.BlockSpec(memory_space=pltpu.SMEM),
        ],
        out_shape=[
            jax.ShapeDtypeStruct((_NCH, _C), jnp.float32),
            jax.ShapeDtypeStruct((1, 1), jnp.float32),
        ],
        scratch_shapes=[
            pltpu.VMEM((_K, _C, _D), jnp.float32),
            pltpu.VMEM((_NCH, _C), jnp.float32),
            pltpu.VMEM((_KO, 1, _C), jnp.float32),
            pltpu.SemaphoreType.DMA((_K,)),
            pltpu.SemaphoreType.DMA,
            pltpu.SemaphoreType.DMA((_KO,)),
            pltpu.SMEM((1,), jnp.float32),
            pltpu.SMEM((1,), jnp.float32),
        ],
    )(parts, wp, w1, b1, w2, b2)


def _norm_step(o_ref, lse_ref, out_ref):
    out_ref[...] = o_ref[...] - lse_ref[0, 0]


def _tc_norm(o, lse):
    return pl.pallas_call(
        _norm_step,
        grid=(_NSTEPS,),
        in_specs=[
            pl.BlockSpec((1, _T), lambda i: (0, i)),
            pl.BlockSpec(memory_space=pltpu.SMEM),
        ],
        out_specs=pl.BlockSpec((1, _T), lambda i: (0, i)),
        out_shape=jax.ShapeDtypeStruct((1, _VOCAB), jnp.float32),
    )(o, lse)


def kernel(x, emb, W_proj, W1, b1, W2, b2):
    x = x.astype(jnp.int32)
    parts = _sc_gather_sum(x, emb)
    o2, lse = _tc_logits(parts, W_proj, W1,
                         b1.reshape(1, _HID), W2, b2.reshape(_NCH, _C))
    return _tc_norm(o2.reshape(1, _VOCAB), lse)
